# baseline (speedup 1.0000x reference)
import functools

import jax
import jax.numpy as jnp
from jax import lax
from jax.experimental import pallas as pl
from jax.experimental.pallas import tpu as pltpu
from jax.experimental.pallas import tpu_sc as plsc

B, L, H, V = 2, 2048, 768, 30522
K, M = 128, 4
N = B * L
VOUT = V + K * (M - 1)
W_COL = 512
NJ = (VOUT + W_COL - 1) // W_COL
VPAD_W = 2 * W_COL
VPAD_OFF = V - (NJ - 2) * W_COL
RB = 512
RB2 = 512

_NC, _NS = 2, 16
_NW = _NC * _NS
_RPW = N // _NW


def _poly_gather_sc(mlm_flat, tok):
    mesh = plsc.VectorSubcoreMesh(core_axis_name="c", subcore_axis_name="s")

    @functools.partial(
        pl.kernel,
        mesh=mesh,
        out_type=jax.ShapeDtypeStruct((N, K), jnp.float32),
        scratch_types=[
            pltpu.VMEM((K,), jnp.int32),
            pltpu.VMEM((_RPW, K), jnp.int32),
            pltpu.VMEM((_RPW, K), jnp.float32),
            pltpu.SemaphoreType.DMA,
        ],
    )
    def gather_kernel(mlm_hbm, tok_hbm, poly_hbm, tok_v, idx_v, out_v, sem):
        wid = lax.axis_index("s") * _NC + lax.axis_index("c")
        base_row = wid * _RPW
        pltpu.sync_copy(tok_hbm, tok_v)

        def compute_idx(i, carry):
            off = (base_row + i) * V
            for t in range(K // 16):
                sl = pl.ds(t * 16, 16)
                idx_v[i, sl] = tok_v[sl] + off
            return carry

        lax.fori_loop(0, _RPW, compute_idx, 0)

        def fire(i, carry):
            pltpu.make_async_copy(
                mlm_hbm.at[idx_v.at[i]], out_v.at[i], sem).start()
            return carry

        lax.fori_loop(0, _RPW, fire, 0)

        def drain(i, carry):
            pltpu.make_async_copy(
                mlm_hbm.at[idx_v.at[i]], out_v.at[i], sem).wait()
            return carry

        lax.fori_loop(0, _RPW, drain, 0)
        pltpu.sync_copy(out_v, poly_hbm.at[pl.ds(base_row, _RPW)])

    return gather_kernel(mlm_flat, tok)


def _gate_body(h_ref, w_ref, p_ref, val0_ref, vpad_ref):
    h = h_ref[...]
    s0 = jnp.dot(h, w_ref[0], preferred_element_type=jnp.float32)
    s1 = jnp.dot(h, w_ref[1], preferred_element_type=jnp.float32)
    s2 = jnp.dot(h, w_ref[2], preferred_element_type=jnp.float32)
    s3 = jnp.dot(h, w_ref[3], preferred_element_type=jnp.float32)
    best = s0
    am = jnp.zeros(s0.shape, jnp.int32)
    for m, sm in ((1, s1), (2, s2), (3, s3)):
        upd = sm > best
        am = jnp.where(upd, m, am)
        best = jnp.where(upd, sm, best)
    p = p_ref[...]
    zero = jnp.zeros_like(p)
    val0_ref[...] = jnp.where(am == 0, p, zero)
    v123 = jnp.concatenate(
        [jnp.where(am == 1, p, zero),
         jnp.where(am == 2, p, zero),
         jnp.where(am == 3, p, zero)], axis=1)
    ii = lax.broadcasted_iota(jnp.int32, (3 * K, VPAD_W), 0)
    jj = lax.broadcasted_iota(jnp.int32, (3 * K, VPAD_W), 1)
    perm = (jj == VPAD_OFF + 3 * (ii % K) + ii // K).astype(jnp.float32)
    vpad_ref[...] = jnp.dot(v123, perm, preferred_element_type=jnp.float32)


def _expand_body(tok_ref, mlm_ref, val0_ref, vpad_ref, out_ref):
    j = pl.program_id(1)
    c0 = j * W_COL
    col = lax.broadcasted_iota(jnp.int32, (RB, W_COL), 1) + c0
    base = jnp.where(col < V, mlm_ref[...], vpad_ref[...])
    scol = lax.broadcasted_iota(jnp.int32, (K, W_COL), 1) + c0
    sel = (scol == tok_ref[...]).astype(jnp.float32)
    hit = jnp.max(sel, axis=0, keepdims=True)
    scat = jnp.dot(val0_ref[...], sel, preferred_element_type=jnp.float32)
    out_ref[...] = base * (1.0 - hit) + scat


def kernel(hidden_states, mlm_logits, W, token_ids):
    hid = hidden_states.reshape(N, H)
    mlm = mlm_logits.reshape(N, V)
    mlm_flat = mlm_logits.reshape(N * V)
    tok = token_ids.astype(jnp.int32)

    poly = _poly_gather_sc(mlm_flat, tok)

    wstack = W.reshape(K, M, H).transpose(1, 2, 0)
    val0, vpad = pl.pallas_call(
        _gate_body,
        grid=(N // RB2,),
        in_specs=[
            pl.BlockSpec((RB2, H), lambda i: (i, 0)),
            pl.BlockSpec((M, H, K), lambda i: (0, 0, 0)),
            pl.BlockSpec((RB2, K), lambda i: (i, 0)),
        ],
        out_specs=[
            pl.BlockSpec((RB2, K), lambda i: (i, 0)),
            pl.BlockSpec((RB2, VPAD_W), lambda i: (i, 0)),
        ],
        out_shape=[
            jax.ShapeDtypeStruct((N, K), jnp.float32),
            jax.ShapeDtypeStruct((N, VPAD_W), jnp.float32),
        ],
        compiler_params=pltpu.CompilerParams(
            dimension_semantics=("parallel",)),
    )(hid, wstack, poly)

    out = pl.pallas_call(
        _expand_body,
        grid=(N // RB, NJ),
        in_specs=[
            pl.BlockSpec((K, 1), lambda i, j: (0, 0)),
            pl.BlockSpec((RB, W_COL), lambda i, j: (i, jnp.minimum(j, NJ - 2))),
            pl.BlockSpec((RB, K), lambda i, j: (i, 0)),
            pl.BlockSpec((RB, W_COL),
                         lambda i, j: (i, jnp.maximum(j - (NJ - 2), 0))),
        ],
        out_specs=pl.BlockSpec((RB, W_COL), lambda i, j: (i, j)),
        out_shape=jax.ShapeDtypeStruct((N, VOUT), jnp.float32),
        compiler_params=pltpu.CompilerParams(
            dimension_semantics=("parallel", "arbitrary")),
    )(tok.reshape(K, 1), mlm, val0, vpad)

    return out.reshape(B, L, VOUT)
